# baseline (speedup 1.0000x reference)
import functools
import math

import jax
import jax.numpy as jnp
from jax import lax
from jax.experimental import pallas as pl
from jax.experimental.pallas import tpu as pltpu
from jax.experimental.pallas import tpu_sc as plsc

D_MODEL = 128
SCALE = math.sqrt(float(D_MODEL))
NUM_CORES = 2
NUM_SUBCORES = 16
NW = NUM_CORES * NUM_SUBCORES
CX = 2
LANES = 16
NBUF = 4
PRE = 2


@functools.partial(jax.jit, static_argnames=("n_rows", "seq"))
def _emb_call(idx3, lut, n_rows, seq):
    rows_per_w = n_rows // NW
    n_chunks = rows_per_w // CX
    assert n_chunks % NBUF == 0

    mesh = plsc.VectorSubcoreMesh(core_axis_name="c", subcore_axis_name="s")

    @functools.partial(
        pl.kernel,
        out_type=jax.ShapeDtypeStruct((n_rows, seq, D_MODEL), jnp.float32),
        mesh=mesh,
        scratch_types=[
            pltpu.VMEM((rows_per_w, seq), jnp.int32),
            pltpu.VMEM((NBUF, CX, seq, D_MODEL), jnp.float32),
            pltpu.SemaphoreType.DMA((NBUF,)),
            pltpu.SemaphoreType.DMA((NBUF,)),
        ],
        compiler_params=pltpu.CompilerParams(use_tc_tiling_on_sc=True),
    )
    def emb(idx_hbm, lut_hbm, out_hbm, idx_v, bufs, gsem, ssem):
        wid = lax.axis_index("s") * NUM_CORES + lax.axis_index("c")
        pltpu.sync_copy(idx_hbm.at[wid], idx_v)
        row_base = wid * rows_per_w

        def gather_copies(j, b):
            return [
                pltpu.make_async_copy(
                    lut_hbm.at[idx_v.at[j * CX + r]], bufs.at[b, r], gsem.at[b]
                )
                for r in range(CX)
            ]

        def scatter_copy(j, b):
            return pltpu.make_async_copy(
                bufs.at[b], out_hbm.at[pl.ds(row_base + j * CX, CX)], ssem.at[b]
            )

        for b in range(PRE):
            for c in gather_copies(b, b):
                c.start()

        def group_body(g, carry):
            for bs in range(NBUF):
                j = g * NBUF + bs
                for c in gather_copies(j, bs):
                    c.wait()

                for xr in range(CX):

                    def row_body(r, c2, xr=xr):
                        for k in range(D_MODEL // LANES):
                            sl = pl.ds(k * LANES, LANES)
                            bufs[bs, xr, r, sl] = bufs[bs, xr, r, sl] * SCALE
                        return c2

                    lax.fori_loop(0, seq, row_body, 0, unroll=2)

                bn = (bs + PRE) % NBUF
                jn = j + PRE

                @pl.when(jn < n_chunks)
                def _():
                    @pl.when(jn >= NBUF)
                    def _():
                        scatter_copy(jn - NBUF, bn).wait()

                    for c in gather_copies(jn, bn):
                        c.start()

                scatter_copy(j, bs).start()
            return carry

        lax.fori_loop(0, n_chunks // NBUF, group_body, 0)

        for bs in range(NBUF):
            scatter_copy(n_chunks - NBUF + bs, bs).wait()

    return emb(idx3, lut)


def kernel(x, lut):
    n_rows, seq = x.shape
    idx3 = x.reshape(NW, n_rows // NW, seq).astype(jnp.int32)
    return _emb_call(idx3, lut, n_rows, seq)


# speedup vs baseline: 1.7800x; 1.7800x over previous
import functools
import math

import jax
import jax.numpy as jnp
from jax import lax
from jax.experimental import pallas as pl
from jax.experimental.pallas import tpu as pltpu
from jax.experimental.pallas import tpu_sc as plsc

D_MODEL = 128
SCALE = math.sqrt(float(D_MODEL))
NUM_CORES = 2
NUM_SUBCORES = 16
NW = NUM_CORES * NUM_SUBCORES
CHUNK = 128
LANES = 16
NBUF = 5
PRE = 2


@functools.partial(jax.jit, static_argnames=("n_chunks",))
def _emb_call(idx, lut, n_chunks):
    B = NW * n_chunks * CHUNK
    per_w = n_chunks * CHUNK
    assert n_chunks % NBUF == 0

    mesh = plsc.VectorSubcoreMesh(core_axis_name="c", subcore_axis_name="s")

    @functools.partial(
        pl.kernel,
        out_type=jax.ShapeDtypeStruct((B, D_MODEL), jnp.float32),
        mesh=mesh,
        scratch_types=[
            pltpu.VMEM((per_w,), jnp.int32),
            pltpu.VMEM((NBUF, CHUNK, D_MODEL), jnp.float32),
            pltpu.SemaphoreType.DMA((NBUF,)),
            pltpu.SemaphoreType.DMA((NBUF,)),
        ],
    )
    def emb(idx_hbm, lut_hbm, out_hbm, idx_v, bufs, gsem, ssem):
        wid = lax.axis_index("s") * NUM_CORES + lax.axis_index("c")
        base = wid * per_w
        pltpu.sync_copy(idx_hbm.at[pl.ds(base, per_w)], idx_v)

        def gather_copy(j, b):
            return pltpu.make_async_copy(
                lut_hbm.at[idx_v.at[pl.ds(j * CHUNK, CHUNK)]],
                bufs.at[b],
                gsem.at[b],
            )

        def scatter_copy(j, b):
            return pltpu.make_async_copy(
                bufs.at[b],
                out_hbm.at[pl.ds(base + j * CHUNK, CHUNK)],
                ssem.at[b],
            )

        for b in range(PRE):
            gather_copy(b, b).start()

        def group_body(g, carry):
            for bs in range(NBUF):
                j = g * NBUF + bs
                gather_copy(j, bs).wait()

                def row_body(r, c2):
                    for k in range(D_MODEL // LANES):
                        sl = pl.ds(k * LANES, LANES)
                        bufs[bs, r, sl] = bufs[bs, r, sl] * SCALE
                    return c2

                lax.fori_loop(0, CHUNK, row_body, 0, unroll=2)

                bn = (bs + PRE) % NBUF
                jn = j + PRE

                @pl.when(jn < n_chunks)
                def _():
                    @pl.when(jn >= NBUF)
                    def _():
                        scatter_copy(jn - NBUF, bn).wait()

                    gather_copy(jn, bn).start()

                scatter_copy(j, bs).start()
            return carry

        lax.fori_loop(0, n_chunks // NBUF, group_body, 0)

        for bs in range(NBUF):
            scatter_copy(n_chunks - NBUF + bs, bs).wait()

    return emb(idx, lut)


def kernel(x, lut):
    n_rows, seq = x.shape
    B = n_rows * seq
    n_chunks = B // (NW * CHUNK)
    idx = jnp.transpose(x).reshape(-1).astype(jnp.int32)
    out = _emb_call(idx, lut, n_chunks)
    return jnp.transpose(out.reshape(seq, n_rows, D_MODEL), (1, 0, 2))
